# baseline (speedup 1.0000x reference)
import math

import jax
import jax.numpy as jnp
from jax import lax
from jax.experimental import pallas as pl

_SMOOTHING = 0.1
_CONFIDENCE = 1.0 - _SMOOTHING
_PAD = 0
_BLOCK_R = 32


def _tc_body(block_r, v, eps, c1):
    def body(target_ref, x_ref, out_ref):
        i = pl.program_id(0)
        xv = x_ref[:, :]
        t = target_ref[:, :]
        cols = lax.broadcasted_iota(jnp.int32, (block_r, v), 1)
        s = jnp.sum(xv, axis=1, keepdims=True)
        g = jnp.sum(jnp.where(cols == t, xv, 0.0), axis=1, keepdims=True)
        x0 = xv[:, 0:1]
        valid = (t != _PAD).astype(jnp.float32)
        per_row = valid * (c1 - eps * s + eps * x0
                           + (eps - _CONFIDENCE) * g)
        partial = jnp.sum(per_row, keepdims=True)

        @pl.when(i == 0)
        def _init():
            out_ref[:, :] = jnp.zeros_like(out_ref)

        out_ref[:, :] += partial

    return body


def kernel(x, target):
    batch, v = x.shape
    eps = _SMOOTHING / (v - 2)
    c1 = eps * math.log(eps) * (v - 2) + _CONFIDENCE * math.log(_CONFIDENCE)
    nblocks = batch // _BLOCK_R

    out = pl.pallas_call(
        _tc_body(_BLOCK_R, v, eps, c1),
        grid=(nblocks,),
        in_specs=[
            pl.BlockSpec((_BLOCK_R, 1), lambda i: (i, 0)),
            pl.BlockSpec((_BLOCK_R, v), lambda i: (i, 0)),
        ],
        out_specs=pl.BlockSpec((1, 1), lambda i: (0, 0)),
        out_shape=jax.ShapeDtypeStruct((1, 1), jnp.float32),
    )(target.astype(jnp.int32).reshape(batch, 1), x)
    return out[0, 0]
